# baseline (speedup 1.0000x reference)
import functools

import jax
import jax.numpy as jnp
import numpy as np
from jax import lax
from jax.experimental import pallas as pl
from jax.experimental.pallas import tpu as pltpu
from jax.experimental.pallas import tpu_sc as plsc

N = 10000
E = 320000
D = 128
ED = 16
H = 128

NC = 2
NS = 16
L = 16
NW = NC * NS
NSEG = 2
ES = E // NSEG
EPW = ES // NW
K = 40
IB = 1000
NBLK = EPW // IB
CPB = IB // K
NP = 10240
RPT = NP // NS
ZR = 8


def _build_colidx():
    idx = np.zeros(D, np.int32)
    for g in range(D // 32):
        o = 32 * g
        for i in range(16):
            idx[o + 2 * i] = o + i
            idx[o + 2 * i + 1] = o + 16 + i
    return idx


_COLIDX = _build_colidx()


BE = 2000


def _mlp_body(ef_ref, w1_ref, b1_ref, w2_ref, b2_ref, w3_ref, b3_ref, c_ref):
    x = jnp.dot(ef_ref[...], w1_ref[...], preferred_element_type=jnp.float32)
    x = x + b1_ref[...]
    x = x * jax.nn.sigmoid(x)
    x = jnp.dot(x.astype(jnp.bfloat16), w2_ref[...],
                preferred_element_type=jnp.float32)
    x = x + b2_ref[...]
    x = x * jax.nn.sigmoid(x)
    w = jnp.dot(x.astype(jnp.bfloat16), w3_ref[...],
                preferred_element_type=jnp.float32)
    c_ref[...] = w + b3_ref[...]


def _edge_mlp(edge_features, W1, b1, W2, b2, W3, b3, interpret=False):
    ne = edge_features.shape[0]
    grid = (ne // BE,)
    return pl.pallas_call(
        _mlp_body,
        grid=grid,
        in_specs=[
            pl.BlockSpec((BE, ED), lambda i: (i, 0)),
            pl.BlockSpec((ED, H), lambda i: (0, 0)),
            pl.BlockSpec((1, H), lambda i: (0, 0)),
            pl.BlockSpec((H, H), lambda i: (0, 0)),
            pl.BlockSpec((1, H), lambda i: (0, 0)),
            pl.BlockSpec((H, D), lambda i: (0, 0)),
            pl.BlockSpec((1, D), lambda i: (0, 0)),
        ],
        out_specs=pl.BlockSpec((BE, D), lambda i: (i, 0)),
        out_shape=jax.ShapeDtypeStruct((ne, D), jnp.float32),
        compiler_params=pltpu.CompilerParams(
            dimension_semantics=("parallel",)),
        interpret=interpret,
    )(edge_features.astype(jnp.bfloat16), W1.astype(jnp.bfloat16),
      b1.reshape(1, H), W2.astype(jnp.bfloat16), b2.reshape(1, H),
      W3.astype(jnp.bfloat16), b3.reshape(1, D))


def _sc_body(h_hbm, src_hbm, dst_hbm, sh_hbm, c_hbm, acc_out, deg_out,
             srcb, dstb, shb, hbuf0, hbuf1, cbuf0, cbuf1, pbuf,
             ones_v, zrow, zdeg, acc_sh, deg_sh, semh0, semh1, semc0, semc1,
             sems, semd):
    cid = lax.axis_index("c")
    sid = lax.axis_index("s")
    wid = sid * NC + cid
    hbufs = (hbuf0, hbuf1)
    cbufs = (cbuf0, cbuf1)
    semh = (semh0, semh1)
    semc = (semc0, semc1)

    def zero_row(j, _):
        zrow[j // 8, pl.ds((j % 8) * L, L)] = jnp.zeros((L,), jnp.float32)
        return 0

    lax.fori_loop(0, ZR * (D // L), zero_row, 0)

    def zero_deg(j, _):
        zdeg[j, :] = jnp.zeros((L,), jnp.float32)
        return 0

    lax.fori_loop(0, ZR, zero_deg, 0)

    def zero_copy(t, _):
        pltpu.sync_copy(zrow, acc_sh.at[pl.ds(sid * RPT + t * ZR, ZR)])
        pltpu.sync_copy(zdeg, deg_sh.at[pl.ds(sid * RPT + t * ZR, ZR)])
        return 0

    lax.fori_loop(0, RPT // ZR, zero_copy, 0)

    def fill_ones(j, _):
        ones_v[j, :] = jnp.ones((L,), jnp.float32)
        return 0

    lax.fori_loop(0, K, fill_ones, 0)

    plsc.subcore_barrier()

    base_w = wid * EPW

    def issue(b, blk, lt):
        sl = pl.ds(lt * K, K)
        pltpu.async_copy(h_hbm.at[srcb.at[sl]], hbufs[b], semh[b])
        gbase = base_w + blk * IB + lt * K
        pltpu.async_copy(c_hbm.at[pl.ds(gbase, K)], cbufs[b], semc[b])

    def wait_scatters():
        pltpu.make_async_copy(pbuf, acc_sh.at[dstb.at[pl.ds(0, K)]],
                              sems).wait()
        pltpu.make_async_copy(ones_v, deg_sh.at[dstb.at[pl.ds(0, K)]],
                              semd).wait()

    def process(b, lt):
        sl = pl.ds(lt * K, K)
        pltpu.make_async_copy(h_hbm.at[srcb.at[sl]], hbufs[b],
                              semh[b]).wait()
        pltpu.make_async_copy(c_hbm.at[pl.ds(0, K)], cbufs[b],
                              semc[b]).wait()

        @pl.when(lt > 0)
        def _():
            wait_scatters()

        hb = hbufs[b]
        cb = cbufs[b]
        sbase = lt * K

        def mulrow(r, _):
            s = shb[pl.ds(sbase + r, L)][0]
            for d in range(D // L):
                o = d * L
                pbuf[r, pl.ds(o, L)] = (hb[r, pl.ds(o, L)]
                                        * cb[r, pl.ds(o, L)] * s)
            return 0

        lax.fori_loop(0, K, mulrow, 0)

        pltpu.async_copy(pbuf, acc_sh.at[dstb.at[sl]], sems, add=True)
        pltpu.async_copy(ones_v, deg_sh.at[dstb.at[sl]], semd, add=True)

    def block(blk, _):
        @pl.when(blk > 0)
        def _():
            wait_scatters()

        bbase = base_w + blk * IB
        pltpu.sync_copy(src_hbm.at[pl.ds(bbase, IB)], srcb)
        pltpu.sync_copy(dst_hbm.at[pl.ds(bbase, IB)], dstb)
        pltpu.sync_copy(sh_hbm.at[pl.ds(bbase, IB)], shb.at[pl.ds(0, IB)])
        issue(0, blk, 0)

        def pair(t2, _):
            issue(1, blk, 2 * t2 + 1)
            process(0, 2 * t2)
            issue(0, blk, 2 * t2 + 2)
            process(1, 2 * t2 + 1)
            return 0

        if CPB % 2 == 0:
            lax.fori_loop(0, (CPB - 2) // 2, pair, 0)
            issue(1, blk, CPB - 1)
            process(0, jnp.int32(CPB - 2))
            process(1, jnp.int32(CPB - 1))
        else:
            lax.fori_loop(0, (CPB - 1) // 2, pair, 0)
            process(0, jnp.int32(CPB - 1))
        return 0

    lax.fori_loop(0, NBLK, block, 0)
    wait_scatters()

    plsc.subcore_barrier()

    r0 = sid * RPT
    pltpu.sync_copy(acc_sh.at[pl.ds(r0, RPT)], acc_out.at[cid, pl.ds(r0, RPT)])
    pltpu.sync_copy(deg_sh.at[pl.ds(r0, RPT)], deg_out.at[cid, pl.ds(r0, RPT)])


def _sc_scatter(h, src, dst, sh, c):
    mesh = plsc.VectorSubcoreMesh(core_axis_name="c", subcore_axis_name="s")
    f = pl.kernel(
        _sc_body,
        out_type=[
            jax.ShapeDtypeStruct((NC, NP, D), jnp.float32),
            jax.ShapeDtypeStruct((NC, NP, L), jnp.float32),
        ],
        mesh=mesh,
        scratch_types=[
            pltpu.VMEM((IB,), jnp.int32),
            pltpu.VMEM((IB,), jnp.int32),
            pltpu.VMEM((IB + L,), jnp.float32),
            pltpu.VMEM((K, D), jnp.float32),
            pltpu.VMEM((K, D), jnp.float32),
            pltpu.VMEM((K, D), jnp.float32),
            pltpu.VMEM((K, D), jnp.float32),
            pltpu.VMEM((K, D), jnp.float32),
            pltpu.VMEM((K, L), jnp.float32),
            pltpu.VMEM((ZR, D), jnp.float32),
            pltpu.VMEM((ZR, L), jnp.float32),
            pltpu.VMEM_SHARED((NP, D), jnp.float32),
            pltpu.VMEM_SHARED((NP, L), jnp.float32),
            pltpu.SemaphoreType.DMA,
            pltpu.SemaphoreType.DMA,
            pltpu.SemaphoreType.DMA,
            pltpu.SemaphoreType.DMA,
            pltpu.SemaphoreType.DMA,
            pltpu.SemaphoreType.DMA,
        ],
        compiler_params=pltpu.CompilerParams(use_tc_tiling_on_sc=False),
    )
    return f(h, src, dst, sh, c)


def _final_body(h_ref, acc_ref, acc2_ref, deg_ref, deg2_ref, ws_ref, wg_ref,
                gamma_ref, beta_ref, out_ref):
    acc = (acc_ref[0] + acc_ref[1]) + (acc2_ref[0] + acc2_ref[1])
    deg = ((deg_ref[0, :, 0:1] + deg_ref[1, :, 0:1])
           + (deg2_ref[0, :, 0:1] + deg2_ref[1, :, 0:1]))
    messages = acc / jnp.maximum(deg, 1.0)
    h = h_ref[...]
    self_update = jnp.dot(h, ws_ref[...], preferred_element_type=jnp.float32)
    update = messages + self_update
    mean = jnp.mean(update, axis=0, keepdims=True)
    var = jnp.mean(jnp.square(update - mean), axis=0, keepdims=True)
    update = (update - mean) * lax.rsqrt(var + 1e-5) * gamma_ref[...]
    update = update + beta_ref[...]
    gate = jax.nn.sigmoid(
        jnp.dot(h, wg_ref[...], preferred_element_type=jnp.float32))
    out_ref[...] = h + gate * update


def _finalize(h, acc_a, acc_b, deg_a, deg_b, W_self, W_gate, gamma, beta,
              interpret=False):
    return pl.pallas_call(
        _final_body,
        out_shape=jax.ShapeDtypeStruct((N, D), jnp.float32),
        interpret=interpret,
    )(h, acc_a, acc_b, deg_a, deg_b, W_self, W_gate, gamma.reshape(1, D),
      beta.reshape(1, D))


def kernel(h, edge_sh, edge_features, graph, W1, b1, W2, b2, W3, b3,
           W_self, W_gate, gamma, beta):
    src = graph[0]
    dst = graph[1]
    sh = edge_sh.reshape(E)
    accs = []
    degs = []
    for s in range(NSEG):
        lo = s * ES
        c = _edge_mlp(edge_features[lo:lo + ES], W1, b1, W2, b2, W3, b3)
        a, d = _sc_scatter(h, src[lo:lo + ES], dst[lo:lo + ES],
                           sh[lo:lo + ES], c)
        accs.append(a)
        degs.append(d)
    return _finalize(h, accs[0][:, :N], accs[1][:, :N], degs[0][:, :N],
                     degs[1][:, :N], W_self, W_gate, gamma, beta)


# speedup vs baseline: 1.0812x; 1.0812x over previous
import functools

import jax
import jax.numpy as jnp
import numpy as np
from jax import lax
from jax.experimental import pallas as pl
from jax.experimental.pallas import tpu as pltpu
from jax.experimental.pallas import tpu_sc as plsc

N = 10000
E = 320000
D = 128
ED = 16
H = 128

NC = 2
NS = 16
L = 16
NW = NC * NS
NSEG = 2
ES = E // NSEG
EPW = ES // NW
K = 40
IB = 1000
NBLK = EPW // IB
CPB = IB // K
NP = 10240
RPT = NP // NS
ZR = 8


def _build_colidx():
    idx = np.zeros(D, np.int32)
    for g in range(D // 32):
        o = 32 * g
        for i in range(16):
            idx[o + 2 * i] = o + i
            idx[o + 2 * i + 1] = o + 16 + i
    return idx


_COLIDX = _build_colidx()


BE = 2000


def _mlp_body(ef_ref, w1_ref, b1_ref, w2_ref, b2_ref, w3_ref, b3_ref, c_ref):
    x = jnp.dot(ef_ref[...], w1_ref[...], preferred_element_type=jnp.float32)
    x = x + b1_ref[...]
    x = x * jax.nn.sigmoid(x)
    x = jnp.dot(x.astype(jnp.bfloat16), w2_ref[...],
                preferred_element_type=jnp.float32)
    x = x + b2_ref[...]
    x = x * jax.nn.sigmoid(x)
    w = jnp.dot(x.astype(jnp.bfloat16), w3_ref[...],
                preferred_element_type=jnp.float32)
    c_ref[...] = w + b3_ref[...]


def _edge_mlp(edge_features, W1, b1, W2, b2, W3, b3, seg=0, interpret=False):
    off = seg * (ES // BE)
    grid = (ES // BE,)
    return pl.pallas_call(
        _mlp_body,
        grid=grid,
        in_specs=[
            pl.BlockSpec((BE, ED), lambda i: (i + off, 0)),
            pl.BlockSpec((ED, H), lambda i: (0, 0)),
            pl.BlockSpec((1, H), lambda i: (0, 0)),
            pl.BlockSpec((H, H), lambda i: (0, 0)),
            pl.BlockSpec((1, H), lambda i: (0, 0)),
            pl.BlockSpec((H, D), lambda i: (0, 0)),
            pl.BlockSpec((1, D), lambda i: (0, 0)),
        ],
        out_specs=pl.BlockSpec((BE, D), lambda i: (i, 0)),
        out_shape=jax.ShapeDtypeStruct((ES, D), jnp.float32),
        compiler_params=pltpu.CompilerParams(
            dimension_semantics=("parallel",)),
        interpret=interpret,
    )(edge_features.astype(jnp.bfloat16), W1.astype(jnp.bfloat16),
      b1.reshape(1, H), W2.astype(jnp.bfloat16), b2.reshape(1, H),
      W3.astype(jnp.bfloat16), b3.reshape(1, D))


def _sc_body(goff, h_hbm, src_hbm, dst_hbm, sh_hbm, c_hbm, acc_out, deg_out,
             srcb, dstb, shb, hbuf0, hbuf1, cbuf0, cbuf1, pbuf,
             ones_v, zrow, zdeg, acc_sh, deg_sh, semh0, semh1, semc0, semc1,
             sems, semd):
    cid = lax.axis_index("c")
    sid = lax.axis_index("s")
    wid = sid * NC + cid
    hbufs = (hbuf0, hbuf1)
    cbufs = (cbuf0, cbuf1)
    semh = (semh0, semh1)
    semc = (semc0, semc1)

    def zero_row(j, _):
        zrow[j // 8, pl.ds((j % 8) * L, L)] = jnp.zeros((L,), jnp.float32)
        return 0

    lax.fori_loop(0, ZR * (D // L), zero_row, 0)

    def zero_deg(j, _):
        zdeg[j, :] = jnp.zeros((L,), jnp.float32)
        return 0

    lax.fori_loop(0, ZR, zero_deg, 0)

    def zero_copy(t, _):
        pltpu.sync_copy(zrow, acc_sh.at[pl.ds(sid * RPT + t * ZR, ZR)])
        pltpu.sync_copy(zdeg, deg_sh.at[pl.ds(sid * RPT + t * ZR, ZR)])
        return 0

    lax.fori_loop(0, RPT // ZR, zero_copy, 0)

    def fill_ones(j, _):
        ones_v[j, :] = jnp.ones((L,), jnp.float32)
        return 0

    lax.fori_loop(0, K, fill_ones, 0)

    plsc.subcore_barrier()

    base_w = wid * EPW

    def issue(b, blk, lt):
        sl = pl.ds(lt * K, K)
        pltpu.async_copy(h_hbm.at[srcb.at[sl]], hbufs[b], semh[b])
        gbase = base_w + blk * IB + lt * K
        pltpu.async_copy(c_hbm.at[pl.ds(gbase, K)], cbufs[b], semc[b])

    def wait_scatters():
        pltpu.make_async_copy(pbuf, acc_sh.at[dstb.at[pl.ds(0, K)]],
                              sems).wait()
        pltpu.make_async_copy(ones_v, deg_sh.at[dstb.at[pl.ds(0, K)]],
                              semd).wait()

    def process(b, lt):
        sl = pl.ds(lt * K, K)
        pltpu.make_async_copy(h_hbm.at[srcb.at[sl]], hbufs[b],
                              semh[b]).wait()
        pltpu.make_async_copy(c_hbm.at[pl.ds(0, K)], cbufs[b],
                              semc[b]).wait()

        @pl.when(lt > 0)
        def _():
            wait_scatters()

        hb = hbufs[b]
        cb = cbufs[b]
        sbase = lt * K

        def mulrow(r, _):
            s = shb[pl.ds(sbase + r, L)][0]
            for d in range(D // L):
                o = d * L
                pbuf[r, pl.ds(o, L)] = (hb[r, pl.ds(o, L)]
                                        * cb[r, pl.ds(o, L)] * s)
            return 0

        lax.fori_loop(0, K, mulrow, 0)

        pltpu.async_copy(pbuf, acc_sh.at[dstb.at[sl]], sems, add=True)
        pltpu.async_copy(ones_v, deg_sh.at[dstb.at[sl]], semd, add=True)

    def block(blk, _):
        @pl.when(blk > 0)
        def _():
            wait_scatters()

        bbase = goff + base_w + blk * IB
        pltpu.sync_copy(src_hbm.at[pl.ds(bbase, IB)], srcb)
        pltpu.sync_copy(dst_hbm.at[pl.ds(bbase, IB)], dstb)
        pltpu.sync_copy(sh_hbm.at[pl.ds(bbase, IB)], shb.at[pl.ds(0, IB)])
        issue(0, blk, 0)

        def pair(t2, _):
            issue(1, blk, 2 * t2 + 1)
            process(0, 2 * t2)
            issue(0, blk, 2 * t2 + 2)
            process(1, 2 * t2 + 1)
            return 0

        if CPB % 2 == 0:
            lax.fori_loop(0, (CPB - 2) // 2, pair, 0)
            issue(1, blk, CPB - 1)
            process(0, jnp.int32(CPB - 2))
            process(1, jnp.int32(CPB - 1))
        else:
            lax.fori_loop(0, (CPB - 1) // 2, pair, 0)
            process(0, jnp.int32(CPB - 1))
        return 0

    lax.fori_loop(0, NBLK, block, 0)
    wait_scatters()

    plsc.subcore_barrier()

    r0 = sid * RPT
    pltpu.sync_copy(acc_sh.at[pl.ds(r0, RPT)], acc_out.at[cid, pl.ds(r0, RPT)])
    pltpu.sync_copy(deg_sh.at[pl.ds(r0, RPT)], deg_out.at[cid, pl.ds(r0, RPT)])


def _sc_scatter(h, src, dst, sh, c, seg=0):
    mesh = plsc.VectorSubcoreMesh(core_axis_name="c", subcore_axis_name="s")
    f = pl.kernel(
        functools.partial(_sc_body, seg * ES),
        out_type=[
            jax.ShapeDtypeStruct((NC, NP, D), jnp.float32),
            jax.ShapeDtypeStruct((NC, NP, L), jnp.float32),
        ],
        mesh=mesh,
        scratch_types=[
            pltpu.VMEM((IB,), jnp.int32),
            pltpu.VMEM((IB,), jnp.int32),
            pltpu.VMEM((IB + L,), jnp.float32),
            pltpu.VMEM((K, D), jnp.float32),
            pltpu.VMEM((K, D), jnp.float32),
            pltpu.VMEM((K, D), jnp.float32),
            pltpu.VMEM((K, D), jnp.float32),
            pltpu.VMEM((K, D), jnp.float32),
            pltpu.VMEM((K, L), jnp.float32),
            pltpu.VMEM((ZR, D), jnp.float32),
            pltpu.VMEM((ZR, L), jnp.float32),
            pltpu.VMEM_SHARED((NP, D), jnp.float32),
            pltpu.VMEM_SHARED((NP, L), jnp.float32),
            pltpu.SemaphoreType.DMA,
            pltpu.SemaphoreType.DMA,
            pltpu.SemaphoreType.DMA,
            pltpu.SemaphoreType.DMA,
            pltpu.SemaphoreType.DMA,
            pltpu.SemaphoreType.DMA,
        ],
        compiler_params=pltpu.CompilerParams(use_tc_tiling_on_sc=False),
    )
    return f(h, src, dst, sh, c)


def _final_body(h_ref, acc_ref, acc2_ref, deg_ref, deg2_ref, ws_ref, wg_ref,
                gamma_ref, beta_ref, out_ref):
    acc = (acc_ref[0] + acc_ref[1]) + (acc2_ref[0] + acc2_ref[1])
    deg = ((deg_ref[0, :, 0:1] + deg_ref[1, :, 0:1])
           + (deg2_ref[0, :, 0:1] + deg2_ref[1, :, 0:1]))
    messages = acc / jnp.maximum(deg, 1.0)
    h = h_ref[...]
    self_update = jnp.dot(h, ws_ref[...], preferred_element_type=jnp.float32)
    update = messages + self_update
    mean = jnp.mean(update, axis=0, keepdims=True)
    var = jnp.mean(jnp.square(update - mean), axis=0, keepdims=True)
    update = (update - mean) * lax.rsqrt(var + 1e-5) * gamma_ref[...]
    update = update + beta_ref[...]
    gate = jax.nn.sigmoid(
        jnp.dot(h, wg_ref[...], preferred_element_type=jnp.float32))
    out_ref[...] = h + gate * update


def _finalize(h, acc_a, acc_b, deg_a, deg_b, W_self, W_gate, gamma, beta,
              interpret=False):
    return pl.pallas_call(
        _final_body,
        out_shape=jax.ShapeDtypeStruct((N, D), jnp.float32),
        interpret=interpret,
    )(h, acc_a, acc_b, deg_a, deg_b, W_self, W_gate, gamma.reshape(1, D),
      beta.reshape(1, D))


def kernel(h, edge_sh, edge_features, graph, W1, b1, W2, b2, W3, b3,
           W_self, W_gate, gamma, beta):
    src = graph[0]
    dst = graph[1]
    sh = edge_sh.reshape(E)
    accs = []
    degs = []
    for s in range(NSEG):
        c = _edge_mlp(edge_features, W1, b1, W2, b2, W3, b3, seg=s)
        a, d = _sc_scatter(h, src, dst, sh, c, seg=s)
        accs.append(a)
        degs.append(d)
    return _finalize(h, accs[0][:, :N], accs[1][:, :N], degs[0][:, :N],
                     degs[1][:, :N], W_self, W_gate, gamma, beta)


# speedup vs baseline: 1.1306x; 1.0457x over previous
import functools

import jax
import jax.numpy as jnp
import numpy as np
from jax import lax
from jax.experimental import pallas as pl
from jax.experimental.pallas import tpu as pltpu
from jax.experimental.pallas import tpu_sc as plsc

N = 10000
E = 320000
D = 128
ED = 16
H = 128

NC = 2
NS = 16
L = 16
NW = NC * NS
NSEG = 2
ES = E // NSEG
EPW = ES // NW
K = 40
IB = 1000
NBLK = EPW // IB
CPB = IB // K
NP = 10240
RPT = NP // NS
ZR = 8


def _build_colidx():
    idx = np.zeros(D, np.int32)
    for g in range(D // 32):
        o = 32 * g
        for i in range(16):
            idx[o + 2 * i] = o + i
            idx[o + 2 * i + 1] = o + 16 + i
    return idx


_COLIDX = _build_colidx()


BE = 1600


def _mlp_body(ef_ref, w1_ref, b1_ref, w2_ref, b2_ref, w3_ref, b3_ref, c_ref):
    x = jnp.dot(ef_ref[...], w1_ref[...], preferred_element_type=jnp.float32)
    x = x + b1_ref[...]
    x = x * jax.nn.sigmoid(x)
    x = jnp.dot(x.astype(jnp.bfloat16), w2_ref[...],
                preferred_element_type=jnp.float32)
    x = x + b2_ref[...]
    x = x * jax.nn.sigmoid(x)
    w = jnp.dot(x.astype(jnp.bfloat16), w3_ref[...],
                preferred_element_type=jnp.float32)
    c_ref[...] = w + b3_ref[...]


def _edge_mlp(edge_features, W1, b1, W2, b2, W3, b3, seg=0, interpret=False):
    off = seg * (ES // BE)
    grid = (ES // BE,)
    return pl.pallas_call(
        _mlp_body,
        grid=grid,
        in_specs=[
            pl.BlockSpec((BE, ED), lambda i: (i + off, 0)),
            pl.BlockSpec((ED, H), lambda i: (0, 0)),
            pl.BlockSpec((1, H), lambda i: (0, 0)),
            pl.BlockSpec((H, H), lambda i: (0, 0)),
            pl.BlockSpec((1, H), lambda i: (0, 0)),
            pl.BlockSpec((H, D), lambda i: (0, 0)),
            pl.BlockSpec((1, D), lambda i: (0, 0)),
        ],
        out_specs=pl.BlockSpec((BE, D), lambda i: (i, 0)),
        out_shape=jax.ShapeDtypeStruct((ES, D), jnp.float32),
        compiler_params=pltpu.CompilerParams(
            dimension_semantics=("parallel",)),
        interpret=interpret,
    )(edge_features.astype(jnp.bfloat16), W1.astype(jnp.bfloat16),
      b1.reshape(1, H), W2.astype(jnp.bfloat16), b2.reshape(1, H),
      W3.astype(jnp.bfloat16), b3.reshape(1, D))


def _sc_body(goff, h_hbm, src_hbm, dst_hbm, sh_hbm, c_hbm, acc_out, deg_out,
             srcb, dstb, shb, hbuf0, hbuf1, cbuf0, cbuf1, pbuf,
             ones_v, zrow, zdeg, acc_sh, deg_sh, semh0, semh1, semc0, semc1,
             sems, semd):
    cid = lax.axis_index("c")
    sid = lax.axis_index("s")
    wid = sid * NC + cid
    hbufs = (hbuf0, hbuf1)
    cbufs = (cbuf0, cbuf1)
    semh = (semh0, semh1)
    semc = (semc0, semc1)

    def zero_row(j, _):
        zrow[j // 8, pl.ds((j % 8) * L, L)] = jnp.zeros((L,), jnp.float32)
        return 0

    lax.fori_loop(0, ZR * (D // L), zero_row, 0)

    def zero_deg(j, _):
        zdeg[j, :] = jnp.zeros((L,), jnp.float32)
        return 0

    lax.fori_loop(0, ZR, zero_deg, 0)

    def zero_copy(t, _):
        pltpu.sync_copy(zrow, acc_sh.at[pl.ds(sid * RPT + t * ZR, ZR)])
        pltpu.sync_copy(zdeg, deg_sh.at[pl.ds(sid * RPT + t * ZR, ZR)])
        return 0

    lax.fori_loop(0, RPT // ZR, zero_copy, 0)

    def fill_ones(j, _):
        ones_v[j, :] = jnp.ones((L,), jnp.float32)
        return 0

    lax.fori_loop(0, K, fill_ones, 0)

    plsc.subcore_barrier()

    base_w = wid * EPW

    def issue(b, blk, lt):
        sl = pl.ds(lt * K, K)
        pltpu.async_copy(h_hbm.at[srcb.at[sl]], hbufs[b], semh[b])
        gbase = base_w + blk * IB + lt * K
        pltpu.async_copy(c_hbm.at[pl.ds(gbase, K)], cbufs[b], semc[b])

    def wait_scatters():
        pltpu.make_async_copy(pbuf, acc_sh.at[dstb.at[pl.ds(0, K)]],
                              sems).wait()
        pltpu.make_async_copy(ones_v, deg_sh.at[dstb.at[pl.ds(0, K)]],
                              semd).wait()

    def process(b, lt):
        sl = pl.ds(lt * K, K)
        pltpu.make_async_copy(h_hbm.at[srcb.at[sl]], hbufs[b],
                              semh[b]).wait()
        pltpu.make_async_copy(c_hbm.at[pl.ds(0, K)], cbufs[b],
                              semc[b]).wait()

        @pl.when(lt > 0)
        def _():
            wait_scatters()

        hb = hbufs[b]
        cb = cbufs[b]
        sbase = lt * K

        def mulrow(r, _):
            s = shb[pl.ds(sbase + r, L)][0]
            for d in range(D // L):
                o = d * L
                pbuf[r, pl.ds(o, L)] = (hb[r, pl.ds(o, L)]
                                        * cb[r, pl.ds(o, L)] * s)
            return 0

        lax.fori_loop(0, K, mulrow, 0)

        pltpu.async_copy(pbuf, acc_sh.at[dstb.at[sl]], sems, add=True)
        pltpu.async_copy(ones_v, deg_sh.at[dstb.at[sl]], semd, add=True)

    def block(blk, _):
        @pl.when(blk > 0)
        def _():
            wait_scatters()

        bbase = goff + base_w + blk * IB
        pltpu.sync_copy(src_hbm.at[pl.ds(bbase, IB)], srcb)
        pltpu.sync_copy(dst_hbm.at[pl.ds(bbase, IB)], dstb)
        pltpu.sync_copy(sh_hbm.at[pl.ds(bbase, IB)], shb.at[pl.ds(0, IB)])
        issue(0, blk, 0)

        def pair(t2, _):
            issue(1, blk, 2 * t2 + 1)
            process(0, 2 * t2)
            issue(0, blk, 2 * t2 + 2)
            process(1, 2 * t2 + 1)
            return 0

        if CPB % 2 == 0:
            lax.fori_loop(0, (CPB - 2) // 2, pair, 0)
            issue(1, blk, CPB - 1)
            process(0, jnp.int32(CPB - 2))
            process(1, jnp.int32(CPB - 1))
        else:
            lax.fori_loop(0, (CPB - 1) // 2, pair, 0)
            process(0, jnp.int32(CPB - 1))
        return 0

    lax.fori_loop(0, NBLK, block, 0)
    wait_scatters()

    plsc.subcore_barrier()

    r0 = sid * RPT
    pltpu.sync_copy(acc_sh.at[pl.ds(r0, RPT)], acc_out.at[cid, pl.ds(r0, RPT)])
    pltpu.sync_copy(deg_sh.at[pl.ds(r0, RPT)], deg_out.at[cid, pl.ds(r0, RPT)])


def _sc_scatter(h, src, dst, sh, c, seg=0):
    mesh = plsc.VectorSubcoreMesh(core_axis_name="c", subcore_axis_name="s")
    f = pl.kernel(
        functools.partial(_sc_body, seg * ES),
        out_type=[
            jax.ShapeDtypeStruct((NC, NP, D), jnp.float32),
            jax.ShapeDtypeStruct((NC, NP, L), jnp.float32),
        ],
        mesh=mesh,
        scratch_types=[
            pltpu.VMEM((IB,), jnp.int32),
            pltpu.VMEM((IB,), jnp.int32),
            pltpu.VMEM((IB + L,), jnp.float32),
            pltpu.VMEM((K, D), jnp.float32),
            pltpu.VMEM((K, D), jnp.float32),
            pltpu.VMEM((K, D), jnp.float32),
            pltpu.VMEM((K, D), jnp.float32),
            pltpu.VMEM((K, D), jnp.float32),
            pltpu.VMEM((K, L), jnp.float32),
            pltpu.VMEM((ZR, D), jnp.float32),
            pltpu.VMEM((ZR, L), jnp.float32),
            pltpu.VMEM_SHARED((NP, D), jnp.float32),
            pltpu.VMEM_SHARED((NP, L), jnp.float32),
            pltpu.SemaphoreType.DMA,
            pltpu.SemaphoreType.DMA,
            pltpu.SemaphoreType.DMA,
            pltpu.SemaphoreType.DMA,
            pltpu.SemaphoreType.DMA,
            pltpu.SemaphoreType.DMA,
        ],
        compiler_params=pltpu.CompilerParams(use_tc_tiling_on_sc=False),
    )
    return f(h, src, dst, sh, c)


def _final_body(h_ref, acc_ref, acc2_ref, deg_ref, deg2_ref, ws_ref, wg_ref,
                gamma_ref, beta_ref, out_ref):
    acc = (acc_ref[0] + acc_ref[1]) + (acc2_ref[0] + acc2_ref[1])
    deg = ((deg_ref[0, :, 0:1] + deg_ref[1, :, 0:1])
           + (deg2_ref[0, :, 0:1] + deg2_ref[1, :, 0:1]))
    messages = acc / jnp.maximum(deg, 1.0)
    h = h_ref[...]
    self_update = jnp.dot(h, ws_ref[...], preferred_element_type=jnp.float32)
    update = messages + self_update
    mean = jnp.mean(update, axis=0, keepdims=True)
    var = jnp.mean(jnp.square(update - mean), axis=0, keepdims=True)
    update = (update - mean) * lax.rsqrt(var + 1e-5) * gamma_ref[...]
    update = update + beta_ref[...]
    gate = jax.nn.sigmoid(
        jnp.dot(h, wg_ref[...], preferred_element_type=jnp.float32))
    out_ref[...] = h + gate * update


def _finalize(h, acc_a, acc_b, deg_a, deg_b, W_self, W_gate, gamma, beta,
              interpret=False):
    return pl.pallas_call(
        _final_body,
        grid=(1,),
        in_specs=[
            pl.BlockSpec((N, D), lambda i: (0, 0)),
            pl.BlockSpec((NC, N, D), lambda i: (0, 0, 0)),
            pl.BlockSpec((NC, N, D), lambda i: (0, 0, 0)),
            pl.BlockSpec((NC, N, L), lambda i: (0, 0, 0)),
            pl.BlockSpec((NC, N, L), lambda i: (0, 0, 0)),
            pl.BlockSpec((D, D), lambda i: (0, 0)),
            pl.BlockSpec((D, D), lambda i: (0, 0)),
            pl.BlockSpec((1, D), lambda i: (0, 0)),
            pl.BlockSpec((1, D), lambda i: (0, 0)),
        ],
        out_specs=pl.BlockSpec((N, D), lambda i: (0, 0)),
        out_shape=jax.ShapeDtypeStruct((N, D), jnp.float32),
        interpret=interpret,
    )(h, acc_a, acc_b, deg_a, deg_b, W_self, W_gate, gamma.reshape(1, D),
      beta.reshape(1, D))


def kernel(h, edge_sh, edge_features, graph, W1, b1, W2, b2, W3, b3,
           W_self, W_gate, gamma, beta):
    src = graph[0]
    dst = graph[1]
    sh = edge_sh.reshape(E)
    accs = []
    degs = []
    for s in range(NSEG):
        c = _edge_mlp(edge_features, W1, b1, W2, b2, W3, b3, seg=s)
        a, d = _sc_scatter(h, src, dst, sh, c, seg=s)
        accs.append(a)
        degs.append(d)
    return _finalize(h, accs[0], accs[1], degs[0], degs[1],
                     W_self, W_gate, gamma, beta)


# speedup vs baseline: 1.1533x; 1.0201x over previous
import functools

import jax
import jax.numpy as jnp
import numpy as np
from jax import lax
from jax.experimental import pallas as pl
from jax.experimental.pallas import tpu as pltpu
from jax.experimental.pallas import tpu_sc as plsc

N = 10000
E = 320000
D = 128
ED = 16
H = 128

NC = 2
NS = 16
L = 16
NW = NC * NS
NSEG = 2
ES = E // NSEG
EPW = ES // NW
K = 40
IB = 1000
NBLK = EPW // IB
CPB = IB // K
NP = 10240
RPT = NP // NS
ZR = 8


def _build_colidx():
    idx = np.zeros(D, np.int32)
    for g in range(D // 32):
        o = 32 * g
        for i in range(16):
            idx[o + 2 * i] = o + i
            idx[o + 2 * i + 1] = o + 16 + i
    return idx


_COLIDX = _build_colidx()


BE = 1600


def _mlp_body(ef_ref, w1_ref, b1_ref, w2_ref, b2_ref, w3_ref, b3_ref, c_ref):
    x = jnp.dot(ef_ref[...], w1_ref[...], preferred_element_type=jnp.float32)
    x = x + b1_ref[...]
    x = x * jax.nn.sigmoid(x)
    x = jnp.dot(x.astype(jnp.bfloat16), w2_ref[...],
                preferred_element_type=jnp.float32)
    x = x + b2_ref[...]
    x = x * jax.nn.sigmoid(x)
    w = jnp.dot(x.astype(jnp.bfloat16), w3_ref[...],
                preferred_element_type=jnp.float32)
    c_ref[...] = w + b3_ref[...]


def _edge_mlp(edge_features, W1, b1, W2, b2, W3, b3, seg=0, interpret=False):
    off = seg * (ES // BE)
    grid = (ES // BE,)
    return pl.pallas_call(
        _mlp_body,
        grid=grid,
        in_specs=[
            pl.BlockSpec((BE, ED), lambda i: (i + off, 0)),
            pl.BlockSpec((ED, H), lambda i: (0, 0)),
            pl.BlockSpec((1, H), lambda i: (0, 0)),
            pl.BlockSpec((H, H), lambda i: (0, 0)),
            pl.BlockSpec((1, H), lambda i: (0, 0)),
            pl.BlockSpec((H, D), lambda i: (0, 0)),
            pl.BlockSpec((1, D), lambda i: (0, 0)),
        ],
        out_specs=pl.BlockSpec((BE, D), lambda i: (i, 0)),
        out_shape=jax.ShapeDtypeStruct((ES, D), jnp.float32),
        compiler_params=pltpu.CompilerParams(
            dimension_semantics=("parallel",)),
        interpret=interpret,
    )(edge_features.astype(jnp.bfloat16), W1.astype(jnp.bfloat16),
      b1.reshape(1, H), W2.astype(jnp.bfloat16), b2.reshape(1, H),
      W3.astype(jnp.bfloat16), b3.reshape(1, D))


def _sc_body(goff, h_hbm, graph_hbm, sh_hbm, c_hbm, acc_out, deg_out,
             srcb, dstb, shb, hbuf0, hbuf1, cbuf0, cbuf1, pbuf,
             ones_v, zrow, zdeg, acc_sh, deg_sh, semh0, semh1, semc0, semc1,
             sems, semd):
    cid = lax.axis_index("c")
    sid = lax.axis_index("s")
    wid = sid * NC + cid
    hbufs = (hbuf0, hbuf1)
    cbufs = (cbuf0, cbuf1)
    semh = (semh0, semh1)
    semc = (semc0, semc1)

    def zero_row(j, _):
        zrow[j // 8, pl.ds((j % 8) * L, L)] = jnp.zeros((L,), jnp.float32)
        return 0

    lax.fori_loop(0, ZR * (D // L), zero_row, 0)

    def zero_deg(j, _):
        zdeg[j, :] = jnp.zeros((L,), jnp.float32)
        return 0

    lax.fori_loop(0, ZR, zero_deg, 0)

    def zero_copy(t, _):
        pltpu.sync_copy(zrow, acc_sh.at[pl.ds(sid * RPT + t * ZR, ZR)])
        pltpu.sync_copy(zdeg, deg_sh.at[pl.ds(sid * RPT + t * ZR, ZR)])
        return 0

    lax.fori_loop(0, RPT // ZR, zero_copy, 0)

    def fill_ones(j, _):
        ones_v[j, :] = jnp.ones((L,), jnp.float32)
        return 0

    lax.fori_loop(0, K, fill_ones, 0)

    plsc.subcore_barrier()

    base_w = wid * EPW

    def issue(b, blk, lt):
        sl = pl.ds(lt * K, K)
        pltpu.async_copy(h_hbm.at[srcb.at[sl]], hbufs[b], semh[b])
        gbase = base_w + blk * IB + lt * K
        pltpu.async_copy(c_hbm.at[pl.ds(gbase, K)], cbufs[b], semc[b])

    def wait_scatters():
        pltpu.make_async_copy(pbuf, acc_sh.at[dstb.at[pl.ds(0, K)]],
                              sems).wait()
        pltpu.make_async_copy(ones_v, deg_sh.at[dstb.at[pl.ds(0, K)]],
                              semd).wait()

    def process(b, lt):
        sl = pl.ds(lt * K, K)
        pltpu.make_async_copy(h_hbm.at[srcb.at[sl]], hbufs[b],
                              semh[b]).wait()
        pltpu.make_async_copy(c_hbm.at[pl.ds(0, K)], cbufs[b],
                              semc[b]).wait()

        @pl.when(lt > 0)
        def _():
            wait_scatters()

        hb = hbufs[b]
        cb = cbufs[b]
        sbase = lt * K

        def mulrow(r, _):
            s = shb[pl.ds(sbase + r, L)][0]
            for d in range(D // L):
                o = d * L
                pbuf[r, pl.ds(o, L)] = (hb[r, pl.ds(o, L)]
                                        * cb[r, pl.ds(o, L)] * s)
            return 0

        lax.fori_loop(0, K, mulrow, 0)

        pltpu.async_copy(pbuf, acc_sh.at[dstb.at[sl]], sems, add=True)
        pltpu.async_copy(ones_v, deg_sh.at[dstb.at[sl]], semd, add=True)

    def block(blk, _):
        @pl.when(blk > 0)
        def _():
            wait_scatters()

        bbase = goff + base_w + blk * IB
        pltpu.sync_copy(graph_hbm.at[0, pl.ds(bbase, IB)], srcb)
        pltpu.sync_copy(graph_hbm.at[1, pl.ds(bbase, IB)], dstb)
        pltpu.sync_copy(sh_hbm.at[pl.ds(bbase, IB)], shb.at[pl.ds(0, IB)])
        issue(0, blk, 0)

        def pair(t2, _):
            issue(1, blk, 2 * t2 + 1)
            process(0, 2 * t2)
            issue(0, blk, 2 * t2 + 2)
            process(1, 2 * t2 + 1)
            return 0

        if CPB % 2 == 0:
            lax.fori_loop(0, (CPB - 2) // 2, pair, 0)
            issue(1, blk, CPB - 1)
            process(0, jnp.int32(CPB - 2))
            process(1, jnp.int32(CPB - 1))
        else:
            lax.fori_loop(0, (CPB - 1) // 2, pair, 0)
            process(0, jnp.int32(CPB - 1))
        return 0

    lax.fori_loop(0, NBLK, block, 0)
    wait_scatters()

    plsc.subcore_barrier()

    r0 = sid * RPT
    pltpu.sync_copy(acc_sh.at[pl.ds(r0, RPT)], acc_out.at[cid, pl.ds(r0, RPT)])
    pltpu.sync_copy(deg_sh.at[pl.ds(r0, RPT)], deg_out.at[cid, pl.ds(r0, RPT)])


def _sc_scatter(h, graph, sh, c, seg=0):
    mesh = plsc.VectorSubcoreMesh(core_axis_name="c", subcore_axis_name="s")
    f = pl.kernel(
        functools.partial(_sc_body, seg * ES),
        out_type=[
            jax.ShapeDtypeStruct((NC, NP, D), jnp.float32),
            jax.ShapeDtypeStruct((NC, NP, L), jnp.float32),
        ],
        mesh=mesh,
        scratch_types=[
            pltpu.VMEM((IB,), jnp.int32),
            pltpu.VMEM((IB,), jnp.int32),
            pltpu.VMEM((IB + L,), jnp.float32),
            pltpu.VMEM((K, D), jnp.float32),
            pltpu.VMEM((K, D), jnp.float32),
            pltpu.VMEM((K, D), jnp.float32),
            pltpu.VMEM((K, D), jnp.float32),
            pltpu.VMEM((K, D), jnp.float32),
            pltpu.VMEM((K, L), jnp.float32),
            pltpu.VMEM((ZR, D), jnp.float32),
            pltpu.VMEM((ZR, L), jnp.float32),
            pltpu.VMEM_SHARED((NP, D), jnp.float32),
            pltpu.VMEM_SHARED((NP, L), jnp.float32),
            pltpu.SemaphoreType.DMA,
            pltpu.SemaphoreType.DMA,
            pltpu.SemaphoreType.DMA,
            pltpu.SemaphoreType.DMA,
            pltpu.SemaphoreType.DMA,
            pltpu.SemaphoreType.DMA,
        ],
        compiler_params=pltpu.CompilerParams(use_tc_tiling_on_sc=False),
    )
    return f(h, graph, sh, c)


def _final_body(h_ref, acc_ref, acc2_ref, deg_ref, deg2_ref, ws_ref, wg_ref,
                gamma_ref, beta_ref, out_ref):
    acc = (acc_ref[0] + acc_ref[1]) + (acc2_ref[0] + acc2_ref[1])
    deg = ((deg_ref[0, :, 0:1] + deg_ref[1, :, 0:1])
           + (deg2_ref[0, :, 0:1] + deg2_ref[1, :, 0:1]))
    messages = acc / jnp.maximum(deg, 1.0)
    h = h_ref[...]
    self_update = jnp.dot(h, ws_ref[...], preferred_element_type=jnp.float32)
    update = messages + self_update
    mean = jnp.mean(update, axis=0, keepdims=True)
    var = jnp.mean(jnp.square(update - mean), axis=0, keepdims=True)
    update = (update - mean) * lax.rsqrt(var + 1e-5) * gamma_ref[...]
    update = update + beta_ref[...]
    gate = jax.nn.sigmoid(
        jnp.dot(h, wg_ref[...], preferred_element_type=jnp.float32))
    out_ref[...] = h + gate * update


def _finalize(h, acc_a, acc_b, deg_a, deg_b, W_self, W_gate, gamma, beta,
              interpret=False):
    return pl.pallas_call(
        _final_body,
        grid=(1,),
        in_specs=[
            pl.BlockSpec((N, D), lambda i: (0, 0)),
            pl.BlockSpec((NC, N, D), lambda i: (0, 0, 0)),
            pl.BlockSpec((NC, N, D), lambda i: (0, 0, 0)),
            pl.BlockSpec((NC, N, L), lambda i: (0, 0, 0)),
            pl.BlockSpec((NC, N, L), lambda i: (0, 0, 0)),
            pl.BlockSpec((D, D), lambda i: (0, 0)),
            pl.BlockSpec((D, D), lambda i: (0, 0)),
            pl.BlockSpec((1, D), lambda i: (0, 0)),
            pl.BlockSpec((1, D), lambda i: (0, 0)),
        ],
        out_specs=pl.BlockSpec((N, D), lambda i: (0, 0)),
        out_shape=jax.ShapeDtypeStruct((N, D), jnp.float32),
        interpret=interpret,
    )(h, acc_a, acc_b, deg_a, deg_b, W_self, W_gate, gamma.reshape(1, D),
      beta.reshape(1, D))


def kernel(h, edge_sh, edge_features, graph, W1, b1, W2, b2, W3, b3,
           W_self, W_gate, gamma, beta):
    sh = edge_sh.reshape(E)
    accs = []
    degs = []
    for s in range(NSEG):
        c = _edge_mlp(edge_features, W1, b1, W2, b2, W3, b3, seg=s)
        a, d = _sc_scatter(h, graph, sh, c, seg=s)
        accs.append(a)
        degs.append(d)
    return _finalize(h, accs[0], accs[1], degs[0], degs[1],
                     W_self, W_gate, gamma, beta)
